# baseline (speedup 1.0000x reference)
import jax
import jax.numpy as jnp
from jax import lax
from jax.experimental import pallas as pl
from jax.experimental.pallas import tpu as pltpu
from jax.experimental.pallas import tpu_sc as plsc

N_USERS = 25000
N_ITEMS = 25000
N_NODES = N_USERS + N_ITEMS
D = 64
HD = D // 2
E = 800000
B = 2048

NC = 2
NS = 16
CHUNK = 128
GCH = 8
GROUP = GCH * CHUNK
GROUPS = 52
NBUF = 4
EPT = GROUPS * GROUP
E_PAD = EPT * NS
ROWS_PT = (N_NODES // NS) // 8 * 8
TAIL = N_NODES - ROWS_PT * NS


def _fused_body(e0v, srcv, dstv2, valv, uemb, i0, i1, users, pos, neg,
                e1o, e2o, e3o, u_o, p_o, n_o,
                ue_o, pe0_o, pe1_o, ne0_o, ne1_o,
                acc,
                sidxb0, sidxb1, drawb0, drawb1, vrawb0, vrawb1,
                rows0, rows1, rows2, rows3, tb, gbuf,
                si0, si1, sg0, sg1, sg2, sg3, ss0, ss1, ss2, ss3):
    cid = lax.axis_index("c")
    sid = lax.axis_index("s")
    coff = cid * N_NODES

    sidxb = (sidxb0, sidxb1)
    drawb = (drawb0, drawb1)
    vrawb = (vrawb0, vrawb1)
    rows = (rows0, rows1, rows2, rows3)
    si = (si0, si1)
    sg = (sg0, sg1, sg2, sg3)
    ss = (ss0, ss1, ss2, ss3)

    zero16 = jnp.zeros((16,), jnp.float32)

    @pl.loop(0, CHUNK)
    def _zero_rows(e):
        for dd in range(HD // 16):
            rows0[e, pl.ds(dd * 16, 16)] = zero16

    rstart = sid * ROWS_PT

    def zero_acc():
        for k in range(ROWS_PT // CHUNK):
            pltpu.sync_copy(rows0, acc.at[pl.ds(rstart + k * CHUNK, CHUNK)])
        rem = ROWS_PT % CHUNK
        if rem:
            pltpu.sync_copy(rows0.at[pl.ds(0, rem)],
                            acc.at[pl.ds(rstart + (ROWS_PT // CHUNK) * CHUNK, rem)])
        if TAIL:
            @pl.when(sid == NS - 1)
            def _zero_tail():
                pltpu.sync_copy(rows0.at[pl.ds(0, TAIL)],
                                acc.at[pl.ds(NS * ROWS_PT, TAIL)])

    ebase = sid * EPT
    crow0 = sid * (EPT // CHUNK)

    def load_group(g, t):
        gb = ebase + g * GROUP
        a = pltpu.async_copy(srcv.at[pl.ds(gb, GROUP)], sidxb[t], si[t])
        b = pltpu.async_copy(dstv2.at[pl.ds(crow0 + g * GCH, GCH)], drawb[t], si[t])
        c = pltpu.async_copy(valv.at[pl.ds(gb, GROUP)], vrawb[t], si[t])
        return (a, b, c)

    def edge_phase(emb):
        def process_group(t):
            @pl.loop(0, GROUP // 16)
            def _rebase(j):
                sl = pl.ds(j * 16, 16)
                sidxb[t][sl] = sidxb[t][sl] + coff

            def gather(c, p):
                return pltpu.async_copy(
                    emb.at[sidxb[t].at[pl.ds(c * CHUNK, CHUNK)]], rows[p], sg[p])

            gd = [None] * NBUF
            sd = [None] * NBUF
            for p in range(NBUF - 1):
                gd[p] = gather(p, p)
            for c in range(GCH):
                p = c % NBUF
                q = (c + NBUF - 1) % NBUF
                if sd[q] is not None:
                    sd[q].wait()
                    sd[q] = None
                if c + NBUF - 1 < GCH:
                    gd[q] = gather(c + NBUF - 1, q)
                gd[p].wait()

                @pl.loop(0, CHUNK // 16)
                def _scale(b):
                    val16 = vrawb[t][pl.ds(c * CHUNK + b * 16, 16)]
                    for e2 in range(16):
                        vv = jnp.full((16,), val16[e2])
                        e = b * 16 + e2
                        for dd in range(HD // 16):
                            csl = pl.ds(dd * 16, 16)
                            rows[p][e, csl] = rows[p][e, csl] * vv

                sd[p] = pltpu.async_copy(rows[p], acc.at[drawb[t].at[c]],
                                         ss[p], add=True)
            for p in range(NBUF):
                if sd[p] is not None:
                    sd[p].wait()

        for d in load_group(0, 0):
            d.wait()

        @pl.loop(0, GROUPS, step=2)
        def _g2(g):
            l1 = load_group(g + 1, 1)
            process_group(0)
            for d in l1:
                d.wait()
            gnxt = jnp.minimum(g + 2, GROUPS - 2)
            l0 = load_group(gnxt, 0)
            process_group(1)
            for d in l0:
                d.wait()

    def writeback(out):
        pltpu.sync_copy(acc.at[pl.ds(rstart, ROWS_PT)],
                        out.at[pl.ds(coff + rstart, ROWS_PT)])
        if TAIL:
            @pl.when(sid == NS - 1)
            def _write_tail():
                pltpu.sync_copy(acc.at[pl.ds(NS * ROWS_PT, TAIL)],
                                out.at[pl.ds(coff + NS * ROWS_PT, TAIL)])

    tables = (e0v, e1o, e2o, e3o)
    for k in range(3):
        zero_acc()
        plsc.subcore_barrier()
        edge_phase(tables[k])
        plsc.subcore_barrier()
        writeback(tables[k + 1])
    plsc.subcore_barrier()

    BPT2 = B // NS
    b2 = sid * BPT2

    def shift_tb(srcb, n, off):
        @pl.loop(0, n // 16)
        def _sh(j):
            sl = pl.ds(j * 16, 16)
            tb[sl] = srcb[sl] + off

    def mean4(idx_hbm, shift, dst):
        pltpu.sync_copy(idx_hbm.at[pl.ds(b2, BPT2)], tb)
        shift_tb(tb, BPT2, coff + shift)
        pltpu.async_copy(tables[0].at[tb], rows0, sg0).wait()
        pltpu.async_copy(tables[1].at[tb], rows0, sg0, add=True).wait()
        pltpu.async_copy(tables[2].at[tb], rows0, sg0, add=True).wait()
        pltpu.async_copy(tables[3].at[tb], rows0, sg0, add=True).wait()
        pltpu.sync_copy(rows0, dst.at[pl.ds(cid * B + b2, BPT2)])

    mean4(users, 0, u_o)
    mean4(pos, N_USERS, p_o)
    mean4(neg, N_USERS, n_o)

    wid = sid * NC + cid
    eb = wid * (B // (NC * NS))

    def ego(table, idx_hbm, dst):
        for h in range(2):
            pltpu.sync_copy(idx_hbm.at[pl.ds(eb + h * 32, 32)],
                            tb.at[pl.ds(0, 32)])
            pltpu.async_copy(table.at[tb.at[pl.ds(0, 32)]], gbuf, sg1).wait()
            pltpu.sync_copy(gbuf, dst.at[pl.ds(eb + h * 32, 32)])

    ego(uemb, users, ue_o)
    ego(i0, pos, pe0_o)
    ego(i1, pos, pe1_o)
    ego(i0, neg, ne0_o)
    ego(i1, neg, ne1_o)


def _make_fused():
    mesh = plsc.VectorSubcoreMesh(core_axis_name="c", subcore_axis_name="s")
    tbl = jax.ShapeDtypeStruct((NC * N_NODES, HD), jnp.float32)
    half2 = jax.ShapeDtypeStruct((NC * B, HD), jnp.float32)
    full = jax.ShapeDtypeStruct((B, D), jnp.float32)
    return pl.kernel(
        _fused_body,
        out_type=(tbl, tbl, tbl, half2, half2, half2,
                  full, full, full, full, full),
        mesh=mesh,
        compiler_params=pltpu.CompilerParams(use_tc_tiling_on_sc=False),
        scratch_types=[
            pltpu.VMEM_SHARED((N_NODES, HD), jnp.float32),
            pltpu.VMEM((GROUP,), jnp.int32),
            pltpu.VMEM((GROUP,), jnp.int32),
            pltpu.VMEM((GCH, CHUNK), jnp.int32),
            pltpu.VMEM((GCH, CHUNK), jnp.int32),
            pltpu.VMEM((GROUP,), jnp.float32),
            pltpu.VMEM((GROUP,), jnp.float32),
            pltpu.VMEM((CHUNK, HD), jnp.float32),
            pltpu.VMEM((CHUNK, HD), jnp.float32),
            pltpu.VMEM((CHUNK, HD), jnp.float32),
            pltpu.VMEM((CHUNK, HD), jnp.float32),
            pltpu.VMEM((CHUNK,), jnp.int32),
            pltpu.VMEM((32, D), jnp.float32),
            pltpu.SemaphoreType.DMA,
            pltpu.SemaphoreType.DMA,
            pltpu.SemaphoreType.DMA,
            pltpu.SemaphoreType.DMA,
            pltpu.SemaphoreType.DMA,
            pltpu.SemaphoreType.DMA,
            pltpu.SemaphoreType.DMA,
            pltpu.SemaphoreType.DMA,
            pltpu.SemaphoreType.DMA,
            pltpu.SemaphoreType.DMA,
        ],
    )


def _items_avg_body(a_ref, b_ref, o_ref):
    o_ref[...] = (a_ref[...] + b_ref[...]) * 0.5


def _items_avg(i0, i1):
    blk = 5000
    return pl.pallas_call(
        _items_avg_body,
        out_shape=jax.ShapeDtypeStruct((N_ITEMS, D), jnp.float32),
        grid=(N_ITEMS // blk,),
        in_specs=[pl.BlockSpec((blk, D), lambda i: (i, 0))] * 2,
        out_specs=pl.BlockSpec((blk, D), lambda i: (i, 0)),
    )(i0, i1)


def _loss_body(u, p, n, ue, pe0, pe1, ne0, ne1, o):
    ps = jnp.sum(u[...] * p[...], axis=1)
    ns = jnp.sum(u[...] * n[...], axis=1)
    ps = ps[:B] + ps[B:]
    ns = ns[:B] + ns[B:]
    x = -(ps - ns) * (1.0 / 16.0)
    sp = jnp.maximum(x, 0.0) + jnp.log1p(jnp.exp(-jnp.abs(x)))
    loss = jnp.sum(sp)
    reg = 0.5 * (
        jnp.sum(ue[...] ** 2)
        + jnp.sum(pe0[...] ** 2)
        + jnp.sum(pe1[...] ** 2)
        + jnp.sum(ne0[...] ** 2)
        + jnp.sum(ne1[...] ** 2)
    ) / float(B)
    row = lax.broadcasted_iota(jnp.int32, (8, 128), 0)
    col = lax.broadcasted_iota(jnp.int32, (8, 128), 1)
    o[...] = jnp.where((row == 0) & (col == 0), loss,
                       jnp.where((row == 0) & (col == 1), reg, 0.0))


def _loss_tc(u, p, n, ue, pe0, pe1, ne0, ne1):
    return pl.pallas_call(
        _loss_body,
        out_shape=jax.ShapeDtypeStruct((8, 128), jnp.float32),
    )(u, p, n, ue, pe0, pe1, ne0, ne1)


@jax.jit
def kernel(user_emb, item_emb0, item_emb1, adj_indices, adj_values, users, pos, neg):
    items_emb = _items_avg(item_emb0, item_emb1)
    e0 = jnp.concatenate([user_emb[:, :HD], items_emb[:, :HD],
                          user_emb[:, HD:], items_emb[:, HD:]], axis=0)

    spread = (jnp.arange(E_PAD - E, dtype=jnp.int32) * 97) % N_NODES
    src = jnp.concatenate([adj_indices[0], spread])
    dst = jnp.concatenate([adj_indices[1], spread])
    dst2 = dst.reshape(E_PAD // CHUNK, CHUNK)
    vals = jnp.concatenate([adj_values, jnp.zeros((E_PAD - E,), jnp.float32)])

    fused = _make_fused()
    (_e1, _e2, _e3, u, p, n, ue, pe0, pe1, ne0, ne1) = fused(
        e0, src, dst2, vals, user_emb, item_emb0, item_emb1, users, pos, neg)

    o = _loss_tc(u, p, n, ue, pe0, pe1, ne0, ne1)
    return jnp.stack([o[0, 0], o[0, 1]])


# speedup vs baseline: 1.0910x; 1.0910x over previous
import jax
import jax.numpy as jnp
from jax import lax
from jax.experimental import pallas as pl
from jax.experimental.pallas import tpu as pltpu
from jax.experimental.pallas import tpu_sc as plsc

N_USERS = 25000
N_ITEMS = 25000
N_NODES = N_USERS + N_ITEMS
D = 64
HD = D // 2
E = 800000
B = 2048

NC = 2
NS = 16
CHUNK = 128
GCH = 16
GROUP = GCH * CHUNK
GROUPS = 26
NBUF = 4
EPT = GROUPS * GROUP
E_PAD = EPT * NS
ROWS_PT = (N_NODES // NS) // 8 * 8
TAIL = N_NODES - ROWS_PT * NS


def _spmm_body(emb, srcv, dstv2, valv, out,
               acc,
               sidxb0, sidxb1, drawb0, drawb1, vrawb0, vrawb1,
               rows0, rows1, rows2, rows3,
               si0, si1, sg0, sg1, sg2, sg3, ss0, ss1, ss2, ss3):
    cid = lax.axis_index("c")
    sid = lax.axis_index("s")
    coff = cid * N_NODES

    sidxb = (sidxb0, sidxb1)
    drawb = (drawb0, drawb1)
    vrawb = (vrawb0, vrawb1)
    rows = (rows0, rows1, rows2, rows3)
    si = (si0, si1)
    sg = (sg0, sg1, sg2, sg3)
    ss = (ss0, ss1, ss2, ss3)

    zero16 = jnp.zeros((16,), jnp.float32)

    @pl.loop(0, CHUNK)
    def _zero_rows(e):
        for dd in range(HD // 16):
            rows0[e, pl.ds(dd * 16, 16)] = zero16

    rstart = sid * ROWS_PT
    for k in range(ROWS_PT // CHUNK):
        pltpu.sync_copy(rows0, acc.at[pl.ds(rstart + k * CHUNK, CHUNK)])
    rem = ROWS_PT % CHUNK
    if rem:
        pltpu.sync_copy(rows0.at[pl.ds(0, rem)],
                        acc.at[pl.ds(rstart + (ROWS_PT // CHUNK) * CHUNK, rem)])
    if TAIL:
        @pl.when(sid == NS - 1)
        def _zero_tail():
            pltpu.sync_copy(rows0.at[pl.ds(0, TAIL)],
                            acc.at[pl.ds(NS * ROWS_PT, TAIL)])

    plsc.subcore_barrier()

    ebase = sid * EPT
    crow0 = sid * (EPT // CHUNK)

    def load_group(g, t):
        gb = ebase + g * GROUP
        a = pltpu.async_copy(srcv.at[pl.ds(gb, GROUP)], sidxb[t], si[t])
        b = pltpu.async_copy(dstv2.at[pl.ds(crow0 + g * GCH, GCH)], drawb[t], si[t])
        c = pltpu.async_copy(valv.at[pl.ds(gb, GROUP)], vrawb[t], si[t])
        return (a, b, c)

    def process_group(t):
        @pl.loop(0, GROUP // 16)
        def _rebase(j):
            sl = pl.ds(j * 16, 16)
            sidxb[t][sl] = sidxb[t][sl] + coff

        def gather(c, p):
            return pltpu.async_copy(
                emb.at[sidxb[t].at[pl.ds(c * CHUNK, CHUNK)]], rows[p], sg[p])

        gd = [None] * NBUF
        sd = [None] * NBUF
        for p in range(NBUF - 1):
            gd[p] = gather(p, p)
        for c in range(GCH):
            p = c % NBUF
            q = (c + NBUF - 1) % NBUF
            if sd[q] is not None:
                sd[q].wait()
                sd[q] = None
            if c + NBUF - 1 < GCH:
                gd[q] = gather(c + NBUF - 1, q)
            gd[p].wait()

            @pl.loop(0, CHUNK // 16)
            def _scale(b):
                val16 = vrawb[t][pl.ds(c * CHUNK + b * 16, 16)]
                for e2 in range(16):
                    vv = jnp.full((16,), val16[e2])
                    e = b * 16 + e2
                    for dd in range(HD // 16):
                        csl = pl.ds(dd * 16, 16)
                        rows[p][e, csl] = rows[p][e, csl] * vv

            sd[p] = pltpu.async_copy(rows[p], acc.at[drawb[t].at[c]],
                                     ss[p], add=True)
        for p in range(NBUF):
            if sd[p] is not None:
                sd[p].wait()

    for d in load_group(0, 0):
        d.wait()

    @pl.loop(0, GROUPS, step=2)
    def _g2(g):
        l1 = load_group(g + 1, 1)
        process_group(0)
        for d in l1:
            d.wait()
        gnxt = jnp.minimum(g + 2, GROUPS - 2)
        l0 = load_group(gnxt, 0)
        process_group(1)
        for d in l0:
            d.wait()

    plsc.subcore_barrier()

    pltpu.sync_copy(acc.at[pl.ds(rstart, ROWS_PT)],
                    out.at[pl.ds(coff + rstart, ROWS_PT)])
    if TAIL:
        @pl.when(sid == NS - 1)
        def _write_tail():
            pltpu.sync_copy(acc.at[pl.ds(NS * ROWS_PT, TAIL)],
                            out.at[pl.ds(coff + NS * ROWS_PT, TAIL)])


def _make_spmm():
    mesh = plsc.VectorSubcoreMesh(core_axis_name="c", subcore_axis_name="s")
    return pl.kernel(
        _spmm_body,
        out_type=jax.ShapeDtypeStruct((NC * N_NODES, HD), jnp.float32),
        mesh=mesh,
        compiler_params=pltpu.CompilerParams(use_tc_tiling_on_sc=False),
        scratch_types=[
            pltpu.VMEM_SHARED((N_NODES, HD), jnp.float32),
            pltpu.VMEM((GROUP,), jnp.int32),
            pltpu.VMEM((GROUP,), jnp.int32),
            pltpu.VMEM((GCH, CHUNK), jnp.int32),
            pltpu.VMEM((GCH, CHUNK), jnp.int32),
            pltpu.VMEM((GROUP,), jnp.float32),
            pltpu.VMEM((GROUP,), jnp.float32),
            pltpu.VMEM((CHUNK, HD), jnp.float32),
            pltpu.VMEM((CHUNK, HD), jnp.float32),
            pltpu.VMEM((CHUNK, HD), jnp.float32),
            pltpu.VMEM((CHUNK, HD), jnp.float32),
            pltpu.SemaphoreType.DMA,
            pltpu.SemaphoreType.DMA,
            pltpu.SemaphoreType.DMA,
            pltpu.SemaphoreType.DMA,
            pltpu.SemaphoreType.DMA,
            pltpu.SemaphoreType.DMA,
            pltpu.SemaphoreType.DMA,
            pltpu.SemaphoreType.DMA,
            pltpu.SemaphoreType.DMA,
            pltpu.SemaphoreType.DMA,
        ],
    )


BPT = B // (NC * NS)


def _batch_gather_body(e0, e1, e2, e3, uemb, i0, i1, users, pos, neg,
                       ua_o, ub_o, pa_o, pb_o, na_o, nb_o,
                       ue_o, pe0_o, pe1_o, ne0_o, ne1_o,
                       ub, pb, nb, tb, abuf, gbuf, sem):
    cid = lax.axis_index("c")
    sid = lax.axis_index("s")
    wid = sid * NC + cid
    base = wid * BPT

    pltpu.sync_copy(users.at[pl.ds(base, BPT)], ub)
    pltpu.sync_copy(pos.at[pl.ds(base, BPT)], pb)
    pltpu.sync_copy(neg.at[pl.ds(base, BPT)], nb)

    def shift(srcb, off):
        @pl.loop(0, BPT // 16)
        def _sh(j):
            sl = pl.ds(j * 16, 16)
            tb[sl] = srcb[sl] + off

    def mean4(dst):
        pltpu.async_copy(e0.at[tb], abuf, sem).wait()
        pltpu.async_copy(e1.at[tb], abuf, sem, add=True).wait()
        pltpu.async_copy(e2.at[tb], abuf, sem, add=True).wait()
        pltpu.async_copy(e3.at[tb], abuf, sem, add=True).wait()
        pltpu.sync_copy(abuf, dst.at[pl.ds(base, BPT)])

    shift(ub, 0)
    mean4(ua_o)
    shift(ub, N_NODES)
    mean4(ub_o)
    shift(pb, N_USERS)
    mean4(pa_o)
    shift(pb, N_NODES + N_USERS)
    mean4(pb_o)
    shift(nb, N_USERS)
    mean4(na_o)
    shift(nb, N_NODES + N_USERS)
    mean4(nb_o)

    def ego(table, idx, dst):
        pltpu.async_copy(table.at[idx], gbuf, sem).wait()
        pltpu.sync_copy(gbuf, dst.at[pl.ds(base, BPT)])

    ego(uemb, ub, ue_o)
    ego(i0, pb, pe0_o)
    ego(i1, pb, pe1_o)
    ego(i0, nb, ne0_o)
    ego(i1, nb, ne1_o)


def _make_batch_gather():
    mesh = plsc.VectorSubcoreMesh(core_axis_name="c", subcore_axis_name="s")
    half = jax.ShapeDtypeStruct((B, HD), jnp.float32)
    full = jax.ShapeDtypeStruct((B, D), jnp.float32)
    return pl.kernel(
        _batch_gather_body,
        out_type=(half,) * 6 + (full,) * 5,
        mesh=mesh,
        compiler_params=pltpu.CompilerParams(use_tc_tiling_on_sc=False),
        scratch_types=[
            pltpu.VMEM((BPT,), jnp.int32),
            pltpu.VMEM((BPT,), jnp.int32),
            pltpu.VMEM((BPT,), jnp.int32),
            pltpu.VMEM((BPT,), jnp.int32),
            pltpu.VMEM((BPT, HD), jnp.float32),
            pltpu.VMEM((BPT, D), jnp.float32),
            pltpu.SemaphoreType.DMA,
        ],
    )


def _items_avg_body(a_ref, b_ref, o_ref):
    o_ref[...] = (a_ref[...] + b_ref[...]) * 0.5


def _items_avg(i0, i1):
    blk = 5000
    return pl.pallas_call(
        _items_avg_body,
        out_shape=jax.ShapeDtypeStruct((N_ITEMS, D), jnp.float32),
        grid=(N_ITEMS // blk,),
        in_specs=[pl.BlockSpec((blk, D), lambda i: (i, 0))] * 2,
        out_specs=pl.BlockSpec((blk, D), lambda i: (i, 0)),
    )(i0, i1)


def _loss_body(ua, ub, pa, pb, na, nb, ue, pe0, pe1, ne0, ne1, o):
    ps = jnp.sum(ua[...] * pa[...], axis=1) + jnp.sum(ub[...] * pb[...], axis=1)
    ns = jnp.sum(ua[...] * na[...], axis=1) + jnp.sum(ub[...] * nb[...], axis=1)
    x = -(ps - ns) * (1.0 / 16.0)
    sp = jnp.maximum(x, 0.0) + jnp.log1p(jnp.exp(-jnp.abs(x)))
    loss = jnp.sum(sp)
    reg = 0.5 * (
        jnp.sum(ue[...] ** 2)
        + jnp.sum(pe0[...] ** 2)
        + jnp.sum(pe1[...] ** 2)
        + jnp.sum(ne0[...] ** 2)
        + jnp.sum(ne1[...] ** 2)
    ) / float(B)
    row = lax.broadcasted_iota(jnp.int32, (8, 128), 0)
    col = lax.broadcasted_iota(jnp.int32, (8, 128), 1)
    o[...] = jnp.where((row == 0) & (col == 0), loss,
                       jnp.where((row == 0) & (col == 1), reg, 0.0))


def _loss_tc(ua, ub, pa, pb, na, nb, ue, pe0, pe1, ne0, ne1):
    return pl.pallas_call(
        _loss_body,
        out_shape=jax.ShapeDtypeStruct((8, 128), jnp.float32),
    )(ua, ub, pa, pb, na, nb, ue, pe0, pe1, ne0, ne1)


@jax.jit
def kernel(user_emb, item_emb0, item_emb1, adj_indices, adj_values, users, pos, neg):
    items_emb = _items_avg(item_emb0, item_emb1)
    e0 = jnp.concatenate([user_emb[:, :HD], items_emb[:, :HD],
                          user_emb[:, HD:], items_emb[:, HD:]], axis=0)

    spread = (jnp.arange(E_PAD - E, dtype=jnp.int32) * 97) % N_NODES
    src = jnp.concatenate([adj_indices[0], spread])
    dst = jnp.concatenate([adj_indices[1], spread])
    dst2 = dst.reshape(E_PAD // CHUNK, CHUNK)
    vals = jnp.concatenate([adj_values, jnp.zeros((E_PAD - E,), jnp.float32)])

    spmm = _make_spmm()
    e1 = spmm(e0, src, dst2, vals)
    e2 = spmm(e1, src, dst2, vals)
    e3 = spmm(e2, src, dst2, vals)

    gather = _make_batch_gather()
    ua, ub, pa, pb, na, nb, ue, pe0, pe1, ne0, ne1 = gather(
        e0, e1, e2, e3, user_emb, item_emb0, item_emb1, users, pos, neg)

    o = _loss_tc(ua, ub, pa, pb, na, nb, ue, pe0, pe1, ne0, ne1)
    return jnp.stack([o[0, 0], o[0, 1]])
